# Initial kernel scaffold; baseline (speedup 1.0000x reference)
import jax
import jax.numpy as jnp
from jax.experimental import pallas as pl


def kernel(features, t, params, g2m_src, g2m_dst, g2m_attr, mm_src, mm_dst, mm_attr, m2g_src, m2g_dst, m2g_attr):
    raise NotImplementedError("write your pallas kernel here")


# speedup vs baseline: 7.7113x; 7.7113x over previous
import functools

import jax
import jax.numpy as jnp
from jax.experimental import pallas as pl

_M_MESH = 50


def _mlp(x, w1, b1, w2, b2, g=None, bt=None):
    h = jnp.dot(x, w1, preferred_element_type=jnp.float32) + b1
    h = jnp.maximum(h, 0.0)
    h = jnp.dot(h, w2, preferred_element_type=jnp.float32) + b2
    if g is not None:
        mu = jnp.mean(h, axis=-1, keepdims=True)
        var = jnp.mean((h - mu) * (h - mu), axis=-1, keepdims=True)
        h = (h - mu) * jax.lax.rsqrt(var + 1e-5) * g + bt
    return h


def _gather_onehot(idx_col, n_rows, n_cols):
    cols = jax.lax.broadcasted_iota(jnp.int32, (n_rows, n_cols), 1)
    return (cols == idx_col).astype(jnp.float32)


def _scatter_onehot_t(idx_row, n_rows, n_cols):
    rows = jax.lax.broadcasted_iota(jnp.int32, (n_rows, n_cols), 0)
    return (rows == idx_row).astype(jnp.float32)


def _body(nb, n_grid, m_mesh, e_g2m, e_mm, e_m2g, feat_dim, *refs):
    (feat_r, g2ma_r, mma_r, m2ga_r,
     g2ms_r, g2md_r, mms_r, mmdc_r, mmdr_r, m2gs_r, m2gd_r) = refs[:11]
    it = iter(refs[11:-1])
    out_r = refs[-1]

    def take6():
        return [next(it) for _ in range(6)]

    enc = take6()
    e_g2m_p = take6()
    e_mesh_p = take6()
    e_m2g_p = take6()
    g2m_e_p = take6()
    g2m_n_p = take6()
    proc_e_p = take6()
    proc_n_p = take6()
    m2g_e_p = take6()
    m2g_n_p = take6()
    dec_p = [next(it) for _ in range(4)]

    def mlp_of(pr, x, ln=True):
        if ln:
            return _mlp(x, pr[0][...], pr[1][...], pr[2][...], pr[3][...],
                        pr[4][...], pr[5][...])
        return _mlp(x, pr[0][...], pr[1][...], pr[2][...], pr[3][...])

    x = feat_r[0]
    grid_h = mlp_of(enc, x)

    ge = mlp_of(e_g2m_p, g2ma_r[...])
    g_gather = _gather_onehot(g2ms_r[...], e_g2m, n_grid)
    msgs = mlp_of(g2m_e_p, jnp.concatenate(
        [jnp.dot(g_gather, grid_h, preferred_element_type=jnp.float32), ge], axis=1))
    s_g2m_t = _scatter_onehot_t(g2md_r[...], m_mesh, e_g2m)
    agg = jnp.dot(s_g2m_t, msgs, preferred_element_type=jnp.float32)
    latent = mlp_of(g2m_n_p, agg)

    me = mlp_of(e_mesh_p, mma_r[...])
    g_src = _gather_onehot(mms_r[...], e_mm, m_mesh)
    g_dst = _gather_onehot(mmdc_r[...], e_mm, m_mesh)
    s_mm_t = _scatter_onehot_t(mmdr_r[...], m_mesh, e_mm)
    for i in range(nb):
        ein = jnp.concatenate(
            [jnp.dot(g_src, latent, preferred_element_type=jnp.float32),
             jnp.dot(g_dst, latent, preferred_element_type=jnp.float32),
             me], axis=1)
        pe = [proc_e_p[0][i], proc_e_p[1][i:i + 1], proc_e_p[2][i],
              proc_e_p[3][i:i + 1], proc_e_p[4][i:i + 1], proc_e_p[5][i:i + 1]]
        me = me + _mlp(ein, *pe)
        agg = jnp.dot(s_mm_t, me, preferred_element_type=jnp.float32)
        pn = [proc_n_p[0][i], proc_n_p[1][i:i + 1], proc_n_p[2][i],
              proc_n_p[3][i:i + 1], proc_n_p[4][i:i + 1], proc_n_p[5][i:i + 1]]
        latent = latent + _mlp(jnp.concatenate([latent, agg], axis=1), *pn)

    de = mlp_of(e_m2g_p, m2ga_r[...])
    g_m2g = _gather_onehot(m2gs_r[...], e_m2g, m_mesh)
    msgs = mlp_of(m2g_e_p, jnp.concatenate(
        [jnp.dot(g_m2g, latent, preferred_element_type=jnp.float32), de], axis=1))
    s_m2g_t = _scatter_onehot_t(m2gd_r[...], n_grid, e_m2g)
    aggn = jnp.dot(s_m2g_t, msgs, preferred_element_type=jnp.float32)
    node_h = mlp_of(m2g_n_p, jnp.concatenate([aggn, grid_h], axis=1))
    out = mlp_of(dec_p, node_h, ln=False) + x[:, :feat_dim]
    out_r[0] = out


def kernel(features, t, params, g2m_src, g2m_dst, g2m_attr,
           mm_src, mm_dst, mm_attr, m2g_src, m2g_dst, m2g_attr):
    del t
    b, n_grid, _ = features.shape
    m_mesh = _M_MESH
    e_g2m = g2m_src.shape[0]
    e_mm = mm_src.shape[0]
    e_m2g = m2g_src.shape[0]
    feat_dim = params['dec']['b2'].shape[0]
    nb = params['proc_e']['W1'].shape[0]

    def flat(d):
        return [d['W1'], d['b1'].reshape(1, -1), d['W2'], d['b2'].reshape(1, -1),
                d['g'].reshape(1, -1), d['bt'].reshape(1, -1)]

    def flat_stacked(d):
        return [d['W1'], d['b1'], d['W2'], d['b2'], d['g'], d['bt']]

    i32 = jnp.int32
    args = [features,
            g2m_attr, mm_attr, m2g_attr,
            g2m_src.astype(i32).reshape(e_g2m, 1),
            g2m_dst.astype(i32).reshape(1, e_g2m),
            mm_src.astype(i32).reshape(e_mm, 1),
            mm_dst.astype(i32).reshape(e_mm, 1),
            mm_dst.astype(i32).reshape(1, e_mm),
            m2g_src.astype(i32).reshape(e_m2g, 1),
            m2g_dst.astype(i32).reshape(1, e_m2g)]
    args += flat(params['enc_node'])
    args += flat(params['e_g2m'])
    args += flat(params['e_mesh'])
    args += flat(params['e_m2g'])
    args += flat(params['g2m_e'])
    args += flat(params['g2m_n'])
    args += flat_stacked(params['proc_e'])
    args += flat_stacked(params['proc_n'])
    args += flat(params['m2g_e'])
    args += flat(params['m2g_n'])
    args += [params['dec']['W1'], params['dec']['b1'].reshape(1, -1),
             params['dec']['W2'], params['dec']['b2'].reshape(1, -1)]

    in_specs = [pl.BlockSpec((1, n_grid, features.shape[2]), lambda bb: (bb, 0, 0))]
    for a in args[1:]:
        r = a.ndim
        in_specs.append(pl.BlockSpec(a.shape, lambda bb, _r=r: (0,) * _r))

    body = functools.partial(_body, nb, n_grid, m_mesh, e_g2m, e_mm, e_m2g, feat_dim)
    return pl.pallas_call(
        body,
        grid=(b,),
        in_specs=in_specs,
        out_specs=pl.BlockSpec((1, n_grid, feat_dim), lambda bb: (bb, 0, 0)),
        out_shape=jax.ShapeDtypeStruct((b, n_grid, feat_dim), jnp.float32),
    )(*args)


# speedup vs baseline: 8.5333x; 1.1066x over previous
import functools

import jax
import jax.numpy as jnp
from jax.experimental import pallas as pl

_M_MESH = 50


def _mlp(x, w1, b1, w2, b2, g=None, bt=None):
    h = jnp.dot(x, w1, preferred_element_type=jnp.float32) + b1
    h = jnp.maximum(h, 0.0)
    h = jnp.dot(h, w2, preferred_element_type=jnp.float32) + b2
    if g is not None:
        mu = jnp.mean(h, axis=-1, keepdims=True)
        var = jnp.mean((h - mu) * (h - mu), axis=-1, keepdims=True)
        h = (h - mu) * jax.lax.rsqrt(var + 1e-5) * g + bt
    return h


def _gather_onehot(idx_col, n_rows, n_cols):
    cols = jax.lax.broadcasted_iota(jnp.int32, (n_rows, n_cols), 1)
    return (cols == idx_col).astype(jnp.float32)


def _scatter_onehot_t(idx_row, n_rows, n_cols):
    rows = jax.lax.broadcasted_iota(jnp.int32, (n_rows, n_cols), 0)
    return (rows == idx_row).astype(jnp.float32)


def _body(nbatch, nb, n_grid, m_mesh, e_g2m, e_mm, e_m2g, feat_dim, *refs):
    (feat_r, g2ma_r, mma_r, m2ga_r,
     g2ms_r, g2md_r, mms_r, mmdc_r, mmdr_r, m2gs_r, m2gd_r) = refs[:11]
    it = iter(refs[11:-1])
    out_r = refs[-1]

    def take6():
        return [next(it) for _ in range(6)]

    enc = take6()
    e_g2m_p = take6()
    e_mesh_p = take6()
    e_m2g_p = take6()
    g2m_e_p = take6()
    g2m_n_p = take6()
    proc_e_p = take6()
    proc_n_p = take6()
    m2g_e_p = take6()
    m2g_n_p = take6()
    dec_p = [next(it) for _ in range(4)]

    def mlp_of(pr, x, ln=True):
        if ln:
            return _mlp(x, pr[0][...], pr[1][...], pr[2][...], pr[3][...],
                        pr[4][...], pr[5][...])
        return _mlp(x, pr[0][...], pr[1][...], pr[2][...], pr[3][...])

    ge = mlp_of(e_g2m_p, g2ma_r[...])
    me0 = mlp_of(e_mesh_p, mma_r[...])
    de = mlp_of(e_m2g_p, m2ga_r[...])
    g_gather = _gather_onehot(g2ms_r[...], e_g2m, n_grid)
    s_g2m_t = _scatter_onehot_t(g2md_r[...], m_mesh, e_g2m)
    g_src = _gather_onehot(mms_r[...], e_mm, m_mesh)
    g_dst = _gather_onehot(mmdc_r[...], e_mm, m_mesh)
    s_mm_t = _scatter_onehot_t(mmdr_r[...], m_mesh, e_mm)
    g_m2g = _gather_onehot(m2gs_r[...], e_m2g, m_mesh)
    s_m2g_t = _scatter_onehot_t(m2gd_r[...], n_grid, e_m2g)

    xs, grid_hs, latents, mes = [], [], [], []
    for b in range(nbatch):
        x = feat_r[b]
        grid_h = mlp_of(enc, x)
        msgs = mlp_of(g2m_e_p, jnp.concatenate(
            [jnp.dot(g_gather, grid_h, preferred_element_type=jnp.float32), ge],
            axis=1))
        agg = jnp.dot(s_g2m_t, msgs, preferred_element_type=jnp.float32)
        xs.append(x)
        grid_hs.append(grid_h)
        latents.append(mlp_of(g2m_n_p, agg))
        mes.append(me0)

    for i in range(nb):
        pe = [proc_e_p[0][i], proc_e_p[1][i:i + 1], proc_e_p[2][i],
              proc_e_p[3][i:i + 1], proc_e_p[4][i:i + 1], proc_e_p[5][i:i + 1]]
        pn = [proc_n_p[0][i], proc_n_p[1][i:i + 1], proc_n_p[2][i],
              proc_n_p[3][i:i + 1], proc_n_p[4][i:i + 1], proc_n_p[5][i:i + 1]]
        for b in range(nbatch):
            latent, me = latents[b], mes[b]
            ein = jnp.concatenate(
                [jnp.dot(g_src, latent, preferred_element_type=jnp.float32),
                 jnp.dot(g_dst, latent, preferred_element_type=jnp.float32),
                 me], axis=1)
            me = me + _mlp(ein, *pe)
            agg = jnp.dot(s_mm_t, me, preferred_element_type=jnp.float32)
            latents[b] = latent + _mlp(jnp.concatenate([latent, agg], axis=1), *pn)
            mes[b] = me

    for b in range(nbatch):
        msgs = mlp_of(m2g_e_p, jnp.concatenate(
            [jnp.dot(g_m2g, latents[b], preferred_element_type=jnp.float32), de],
            axis=1))
        aggn = jnp.dot(s_m2g_t, msgs, preferred_element_type=jnp.float32)
        node_h = mlp_of(m2g_n_p, jnp.concatenate([aggn, grid_hs[b]], axis=1))
        out = mlp_of(dec_p, node_h, ln=False) + xs[b][:, :feat_dim]
        out_r[b] = out


def kernel(features, t, params, g2m_src, g2m_dst, g2m_attr,
           mm_src, mm_dst, mm_attr, m2g_src, m2g_dst, m2g_attr):
    del t
    b, n_grid, _ = features.shape
    m_mesh = _M_MESH
    e_g2m = g2m_src.shape[0]
    e_mm = mm_src.shape[0]
    e_m2g = m2g_src.shape[0]
    feat_dim = params['dec']['b2'].shape[0]
    nb = params['proc_e']['W1'].shape[0]

    def flat(d):
        return [d['W1'], d['b1'].reshape(1, -1), d['W2'], d['b2'].reshape(1, -1),
                d['g'].reshape(1, -1), d['bt'].reshape(1, -1)]

    def flat_stacked(d):
        return [d['W1'], d['b1'], d['W2'], d['b2'], d['g'], d['bt']]

    i32 = jnp.int32
    args = [features,
            g2m_attr, mm_attr, m2g_attr,
            g2m_src.astype(i32).reshape(e_g2m, 1),
            g2m_dst.astype(i32).reshape(1, e_g2m),
            mm_src.astype(i32).reshape(e_mm, 1),
            mm_dst.astype(i32).reshape(e_mm, 1),
            mm_dst.astype(i32).reshape(1, e_mm),
            m2g_src.astype(i32).reshape(e_m2g, 1),
            m2g_dst.astype(i32).reshape(1, e_m2g)]
    args += flat(params['enc_node'])
    args += flat(params['e_g2m'])
    args += flat(params['e_mesh'])
    args += flat(params['e_m2g'])
    args += flat(params['g2m_e'])
    args += flat(params['g2m_n'])
    args += flat_stacked(params['proc_e'])
    args += flat_stacked(params['proc_n'])
    args += flat(params['m2g_e'])
    args += flat(params['m2g_n'])
    args += [params['dec']['W1'], params['dec']['b1'].reshape(1, -1),
             params['dec']['W2'], params['dec']['b2'].reshape(1, -1)]

    body = functools.partial(_body, b, nb, n_grid, m_mesh, e_g2m, e_mm, e_m2g,
                             feat_dim)
    return pl.pallas_call(
        body,
        out_shape=jax.ShapeDtypeStruct((b, n_grid, feat_dim), jnp.float32),
    )(*args)


# speedup vs baseline: 8.5718x; 1.0045x over previous
import functools

import jax
import jax.numpy as jnp
from jax.experimental import pallas as pl

_M_MESH = 50


def _mlp(x, w1, b1, w2, b2, g=None, bt=None):
    h = jnp.dot(x, w1, preferred_element_type=jnp.float32) + b1
    h = jnp.maximum(h, 0.0)
    h = jnp.dot(h, w2, preferred_element_type=jnp.float32) + b2
    if g is not None:
        mu = jnp.mean(h, axis=-1, keepdims=True)
        var = jnp.mean((h - mu) * (h - mu), axis=-1, keepdims=True)
        h = (h - mu) * jax.lax.rsqrt(var + 1e-5) * g + bt
    return h


def _gather_onehot(idx_col, n_rows, n_cols):
    cols = jax.lax.broadcasted_iota(jnp.int32, (n_rows, n_cols), 1)
    return (cols == idx_col).astype(jnp.float32)


def _scatter_onehot_t(idx_row, n_rows, n_cols):
    rows = jax.lax.broadcasted_iota(jnp.int32, (n_rows, n_cols), 0)
    return (rows == idx_row).astype(jnp.float32)


def _body(nbatch, nb, n_grid, m_mesh, e_g2m, e_mm, e_m2g, feat_dim, *refs):
    (feat_r, g2ma_r, mma_r, m2ga_r,
     g2ms_r, g2md_r, mms_r, mmdc_r, mmdr_r, m2gs_r, m2gd_r) = refs[:11]
    it = iter(refs[11:-1])
    out_r = refs[-1]

    def take6():
        return [next(it) for _ in range(6)]

    enc = take6()
    e_g2m_p = take6()
    e_mesh_p = take6()
    e_m2g_p = take6()
    g2m_e_p = take6()
    g2m_n_p = take6()
    proc_e_p = take6()
    proc_n_p = take6()
    m2g_e_p = take6()
    m2g_n_p = take6()
    dec_p = [next(it) for _ in range(4)]

    def mlp_of(pr, x, ln=True):
        if ln:
            return _mlp(x, pr[0][...], pr[1][...], pr[2][...], pr[3][...],
                        pr[4][...], pr[5][...])
        return _mlp(x, pr[0][...], pr[1][...], pr[2][...], pr[3][...])

    ge = mlp_of(e_g2m_p, g2ma_r[...])
    me0 = mlp_of(e_mesh_p, mma_r[...])
    de = mlp_of(e_m2g_p, m2ga_r[...])
    g_gather = _gather_onehot(g2ms_r[...], e_g2m, n_grid)
    s_g2m_t = _scatter_onehot_t(g2md_r[...], m_mesh, e_g2m)
    g_src = _gather_onehot(mms_r[...], e_mm, m_mesh)
    g_dst = _gather_onehot(mmdc_r[...], e_mm, m_mesh)
    s_mm_t = _scatter_onehot_t(mmdr_r[...], m_mesh, e_mm)
    g_m2g = _gather_onehot(m2gs_r[...], e_m2g, m_mesh)
    s_m2g_t = _scatter_onehot_t(m2gd_r[...], n_grid, e_m2g)

    xs, grid_hs, latents, mes = [], [], [], []
    for b in range(nbatch):
        x = feat_r[b]
        grid_h = mlp_of(enc, x)
        msgs = mlp_of(g2m_e_p, jnp.concatenate(
            [jnp.dot(g_gather, grid_h, preferred_element_type=jnp.float32), ge],
            axis=1))
        agg = jnp.dot(s_g2m_t, msgs, preferred_element_type=jnp.float32)
        xs.append(x)
        grid_hs.append(grid_h)
        latents.append(mlp_of(g2m_n_p, agg))
        mes.append(me0)

    for i in range(nb):
        pe = [proc_e_p[0][i], proc_e_p[1][i], proc_e_p[2][i],
              proc_e_p[3][i], proc_e_p[4][i], proc_e_p[5][i]]
        pn = [proc_n_p[0][i], proc_n_p[1][i], proc_n_p[2][i],
              proc_n_p[3][i], proc_n_p[4][i], proc_n_p[5][i]]
        for b in range(nbatch):
            latent, me = latents[b], mes[b]
            ein = jnp.concatenate(
                [jnp.dot(g_src, latent, preferred_element_type=jnp.float32),
                 jnp.dot(g_dst, latent, preferred_element_type=jnp.float32),
                 me], axis=1)
            me = me + _mlp(ein, *pe)
            agg = jnp.dot(s_mm_t, me, preferred_element_type=jnp.float32)
            latents[b] = latent + _mlp(jnp.concatenate([latent, agg], axis=1), *pn)
            mes[b] = me

    for b in range(nbatch):
        msgs = mlp_of(m2g_e_p, jnp.concatenate(
            [jnp.dot(g_m2g, latents[b], preferred_element_type=jnp.float32), de],
            axis=1))
        aggn = jnp.dot(s_m2g_t, msgs, preferred_element_type=jnp.float32)
        node_h = mlp_of(m2g_n_p, jnp.concatenate([aggn, grid_hs[b]], axis=1))
        out = mlp_of(dec_p, node_h, ln=False) + xs[b][:, :feat_dim]
        out_r[b] = out


def kernel(features, t, params, g2m_src, g2m_dst, g2m_attr,
           mm_src, mm_dst, mm_attr, m2g_src, m2g_dst, m2g_attr):
    del t
    b, n_grid, _ = features.shape
    m_mesh = _M_MESH
    e_g2m = g2m_src.shape[0]
    e_mm = mm_src.shape[0]
    e_m2g = m2g_src.shape[0]
    feat_dim = params['dec']['b2'].shape[0]
    nb = params['proc_e']['W1'].shape[0]

    def flat(d):
        return [d['W1'], d['b1'], d['W2'], d['b2'], d['g'], d['bt']]

    i32 = jnp.int32
    args = [features, g2m_attr, mm_attr, m2g_attr,
            g2m_src.astype(i32).reshape(e_g2m, 1),
            g2m_dst.astype(i32).reshape(1, e_g2m),
            mm_src.astype(i32).reshape(e_mm, 1),
            mm_dst.astype(i32).reshape(e_mm, 1),
            mm_dst.astype(i32).reshape(1, e_mm),
            m2g_src.astype(i32).reshape(e_m2g, 1),
            m2g_dst.astype(i32).reshape(1, e_m2g)]
    args += flat(params['enc_node'])
    args += flat(params['e_g2m'])
    args += flat(params['e_mesh'])
    args += flat(params['e_m2g'])
    args += flat(params['g2m_e'])
    args += flat(params['g2m_n'])
    args += flat(params['proc_e'])
    args += flat(params['proc_n'])
    args += flat(params['m2g_e'])
    args += flat(params['m2g_n'])
    args += [params['dec']['W1'], params['dec']['b1'],
             params['dec']['W2'], params['dec']['b2']]

    body = functools.partial(_body, b, nb, n_grid, m_mesh, e_g2m, e_mm, e_m2g,
                             feat_dim)
    return pl.pallas_call(
        body,
        out_shape=jax.ShapeDtypeStruct((b, n_grid, feat_dim), jnp.float32),
    )(*args)
